# baseline (speedup 1.0000x reference)
import functools

import jax
import jax.numpy as jnp
import numpy as np
from jax.experimental import pallas as pl
from jax.experimental.pallas import tpu as pltpu
from jax.experimental.shard_map import shard_map
from jax.sharding import Mesh, PartitionSpec as P

N = 10000
NFEAT = 128
NCLASS = 16
ALPHA = 0.2
LOG2E = 1.4426950408889634

_DEVS = jax.devices()
NDEV = 2 if (len(_DEVS) >= 2 and N % 2 == 0) else 1
MESH = Mesh(np.array(_DEVS[:NDEV]), ("r",))
NLOC = N // NDEV

BM = 512
NI = pl.cdiv(NLOC, BM)
BMP = 512
NIP = pl.cdiv(N, BMP)


def _prologue_body(x_ref, watt_ref, a_ref, wres_ref,
                   wh_ref, res_ref, src_ref, d_ref, w_ref):
    x = x_ref[:, :]
    wh = jnp.dot(x, watt_ref[:, :], preferred_element_type=jnp.float32)
    wh_ref[:, :NCLASS] = wh.astype(jnp.bfloat16)
    wh_ref[:, NCLASS:] = jnp.ones((BMP, 1), jnp.bfloat16)

    res_ref[:, :] = jnp.dot(x, wres_ref[:, :],
                            preferred_element_type=jnp.float32)

    a = a_ref[:, :]
    src_ref[:, :] = jnp.sum(wh * a[:, :NCLASS], axis=1, keepdims=True)

    dst_row = jax.lax.dot_general(
        a[:, NCLASS:], wh, (((1,), (1,)), ((), ())),
        preferred_element_type=jnp.float32)
    d_ref[:, :] = LOG2E * dst_row
    w_ref[:, :] = (ALPHA * LOG2E) * dst_row


def _main_body(adj_ref, wh_ref, src_ref, d_ref, w_ref, res_ref, out_ref):
    dmax2 = jnp.max(d_ref[:, :])
    src2 = LOG2E * src_ref[:, :]
    etop = src2 + dmax2
    c2 = jnp.maximum(etop, ALPHA * etop)
    u = src2 - c2
    v = ALPHA * src2 - c2

    s2 = jnp.maximum(u + d_ref[:, :], v + w_ref[:, :])
    p = (adj_ref[:, :] * jnp.exp2(s2)).astype(jnp.bfloat16)
    acc = jnp.dot(p, wh_ref[:, :], preferred_element_type=jnp.float32)

    h = acc[:, :NCLASS] / acc[:, NCLASS:]
    h = jnp.where(h > 0, h, jnp.exp(h) - 1.0)
    o = h + res_ref[:, :]
    mx = jnp.max(o, axis=1, keepdims=True)
    lse = mx + jnp.log(jnp.sum(jnp.exp(o - mx), axis=1, keepdims=True))
    out_ref[:, :] = o - lse


def _shard_fn(raw_x, adj, W_att, a_row, W_res):
    wh, res, src, d2, w2 = pl.pallas_call(
        _prologue_body,
        grid=(NIP,),
        in_specs=[
            pl.BlockSpec((BMP, NFEAT), lambda i: (i, 0)),
            pl.BlockSpec((NFEAT, NCLASS), lambda i: (0, 0)),
            pl.BlockSpec((1, 2 * NCLASS), lambda i: (0, 0)),
            pl.BlockSpec((NFEAT, NCLASS), lambda i: (0, 0)),
        ],
        out_specs=[
            pl.BlockSpec((BMP, NCLASS + 1), lambda i: (i, 0)),
            pl.BlockSpec((BMP, NCLASS), lambda i: (i, 0)),
            pl.BlockSpec((BMP, 1), lambda i: (i, 0)),
            pl.BlockSpec((1, BMP), lambda i: (0, i)),
            pl.BlockSpec((1, BMP), lambda i: (0, i)),
        ],
        out_shape=[
            jax.ShapeDtypeStruct((N, NCLASS + 1), jnp.bfloat16),
            jax.ShapeDtypeStruct((N, NCLASS), jnp.float32),
            jax.ShapeDtypeStruct((N, 1), jnp.float32),
            jax.ShapeDtypeStruct((1, N), jnp.float32),
            jax.ShapeDtypeStruct((1, N), jnp.float32),
        ],
    )(raw_x, W_att, a_row, W_res)

    base = jax.lax.axis_index("r") * NLOC
    src_loc = jax.lax.dynamic_slice_in_dim(src, base, NLOC, 0)
    res_loc = jax.lax.dynamic_slice_in_dim(res, base, NLOC, 0)

    out = pl.pallas_call(
        _main_body,
        grid=(NI,),
        in_specs=[
            pl.BlockSpec((BM, N), lambda i: (i, 0)),
            pl.BlockSpec((N, NCLASS + 1), lambda i: (0, 0)),
            pl.BlockSpec((BM, 1), lambda i: (i, 0)),
            pl.BlockSpec((1, N), lambda i: (0, 0)),
            pl.BlockSpec((1, N), lambda i: (0, 0)),
            pl.BlockSpec((BM, NCLASS), lambda i: (i, 0)),
        ],
        out_specs=pl.BlockSpec((BM, NCLASS), lambda i: (i, 0)),
        out_shape=jax.ShapeDtypeStruct((NLOC, NCLASS), jnp.float32),
        compiler_params=pltpu.CompilerParams(
            dimension_semantics=("arbitrary",),
        ),
    )(adj, wh, src_loc, d2, w2, res_loc)

    return out


@jax.jit
def kernel(raw_x, adj, W_att, a_att, W_res):
    a_row = a_att.reshape(1, 2 * NCLASS)
    f = shard_map(
        _shard_fn, mesh=MESH,
        in_specs=(P(), P("r", None), P(), P(), P()),
        out_specs=P("r", None),
        check_rep=False,
    )
    return f(raw_x, adj, W_att, a_row, W_res)


# speedup vs baseline: 5.1517x; 5.1517x over previous
import jax
import jax.numpy as jnp
from jax.experimental import pallas as pl
from jax.experimental.pallas import tpu as pltpu

N = 10000
NFEAT = 128
NCLASS = 16
ALPHA = 0.2
LOG2E = 1.4426950408889634

BM = 512
NI = pl.cdiv(N, BM)
BMP = 2048
NIP = pl.cdiv(N, BMP)


def _prologue_body(x_ref, watt_ref, a_ref, wres_ref,
                   wh_ref, res_ref, src_ref, d_ref, w_ref):
    x = x_ref[:, :]
    wh = jnp.dot(x, watt_ref[:, :], preferred_element_type=jnp.float32)
    wh_ref[:, :NCLASS] = wh.astype(jnp.bfloat16)
    wh_ref[:, NCLASS:] = jnp.ones((BMP, 1), jnp.bfloat16)

    res_ref[:, :] = jnp.dot(x, wres_ref[:, :],
                            preferred_element_type=jnp.float32)

    a = a_ref[:, :]
    src_ref[:, :] = jnp.sum(wh * a[:, :NCLASS], axis=1, keepdims=True)

    dst_row = jax.lax.dot_general(
        a[:, NCLASS:], wh, (((1,), (1,)), ((), ())),
        preferred_element_type=jnp.float32)
    d_ref[:, :] = LOG2E * dst_row
    w_ref[:, :] = (ALPHA * LOG2E) * dst_row


def _main_body(adj_ref, wh_ref, src_ref, d_ref, w_ref, res_ref, out_ref):
    dmax2 = jnp.max(d_ref[:, :])
    src2 = LOG2E * src_ref[:, :]
    etop = src2 + dmax2
    c2 = jnp.maximum(etop, ALPHA * etop)
    u = src2 - c2
    v = ALPHA * src2 - c2

    s2 = jnp.maximum(u + d_ref[:, :], v + w_ref[:, :])
    p = (adj_ref[:, :] * jnp.exp2(s2)).astype(jnp.bfloat16)
    acc = jnp.dot(p, wh_ref[:, :], preferred_element_type=jnp.float32)

    h = acc[:, :NCLASS] / acc[:, NCLASS:]
    h = jnp.where(h > 0, h, jnp.exp(h) - 1.0)
    o = h + res_ref[:, :]
    mx = jnp.max(o, axis=1, keepdims=True)
    lse = mx + jnp.log(jnp.sum(jnp.exp(o - mx), axis=1, keepdims=True))
    out_ref[:, :] = o - lse


@jax.jit
def kernel(raw_x, adj, W_att, a_att, W_res):
    a_row = a_att.reshape(1, 2 * NCLASS)

    wh, res, src, d2, w2 = pl.pallas_call(
        _prologue_body,
        grid=(NIP,),
        in_specs=[
            pl.BlockSpec((BMP, NFEAT), lambda i: (i, 0)),
            pl.BlockSpec((NFEAT, NCLASS), lambda i: (0, 0)),
            pl.BlockSpec((1, 2 * NCLASS), lambda i: (0, 0)),
            pl.BlockSpec((NFEAT, NCLASS), lambda i: (0, 0)),
        ],
        out_specs=[
            pl.BlockSpec((BMP, NCLASS + 1), lambda i: (i, 0)),
            pl.BlockSpec((BMP, NCLASS), lambda i: (i, 0)),
            pl.BlockSpec((BMP, 1), lambda i: (i, 0)),
            pl.BlockSpec((1, BMP), lambda i: (0, i)),
            pl.BlockSpec((1, BMP), lambda i: (0, i)),
        ],
        out_shape=[
            jax.ShapeDtypeStruct((N, NCLASS + 1), jnp.bfloat16),
            jax.ShapeDtypeStruct((N, NCLASS), jnp.float32),
            jax.ShapeDtypeStruct((N, 1), jnp.float32),
            jax.ShapeDtypeStruct((1, N), jnp.float32),
            jax.ShapeDtypeStruct((1, N), jnp.float32),
        ],
    )(raw_x, W_att, a_row, W_res)

    out = pl.pallas_call(
        _main_body,
        grid=(NI,),
        in_specs=[
            pl.BlockSpec((BM, N), lambda i: (i, 0)),
            pl.BlockSpec((N, NCLASS + 1), lambda i: (0, 0)),
            pl.BlockSpec((BM, 1), lambda i: (i, 0)),
            pl.BlockSpec((1, N), lambda i: (0, 0)),
            pl.BlockSpec((1, N), lambda i: (0, 0)),
            pl.BlockSpec((BM, NCLASS), lambda i: (i, 0)),
        ],
        out_specs=pl.BlockSpec((BM, NCLASS), lambda i: (i, 0)),
        out_shape=jax.ShapeDtypeStruct((N, NCLASS), jnp.float32),
        compiler_params=pltpu.CompilerParams(
            dimension_semantics=("arbitrary",),
        ),
    )(adj, wh, src, d2, w2, res)

    return out
